# Initial kernel scaffold; baseline (speedup 1.0000x reference)
import jax
import jax.numpy as jnp
from jax.experimental import pallas as pl


def kernel(x, cg_w1, cg_b1, cg_w2, cg_b2, fg_w, fg_b, ex_w1, ex_b1, ex_w2, ex_b2, ex_g, ex_beta, rh_w, rh_b, ch_w, ch_b):
    raise NotImplementedError("write your pallas kernel here")


# speedup vs baseline: 3.1777x; 3.1777x over previous
import functools
import math

import jax
import jax.numpy as jnp
from jax import lax
from jax.experimental import pallas as pl
from jax.experimental.pallas import tpu as pltpu

B = 2048
IN_DIM = 1024
N_SUPER = 2
N_SUB = 2
E = 4
HID = 2048
OUT = 512
AUX_COEF = 0.01

BT = 256
NT = B // BT

_HI = lax.Precision.HIGHEST


def _gating_body(x_ref, cgw1_ref, cgb1_ref, cgw2_ref, cgb2_ref,
                 fgw_ref, fgb_ref,
                 leaf_ref, aux_ref, acc_ref):
    i = pl.program_id(0)
    x = x_ref[...]
    h = lax.dot_general(x, cgw1_ref[...], (((1,), (1,)), ((), ())),
                        preferred_element_type=jnp.float32)
    h = jnp.maximum(h + cgb1_ref[...], 0.0)
    cl = lax.dot_general(h, cgw2_ref[...], (((1,), (1,)), ((), ())),
                         preferred_element_type=jnp.float32)
    cl = cl + cgb2_ref[...]
    m = jnp.max(cl, axis=1, keepdims=True)
    ex = jnp.exp(cl - m)
    cw = ex / jnp.sum(ex, axis=1, keepdims=True)
    ohc0 = (cl[:, 0:1] >= cl[:, 1:2]).astype(jnp.float32)

    x_aug = jnp.concatenate([x, cw], axis=1)
    fl = (lax.dot_general(x_aug, fgw_ref[...], (((1,), (1,)), ((), ())),
                          preferred_element_type=jnp.float32)
          + fgb_ref[...])
    oh0 = (fl[:, 0:1] >= fl[:, 1:2]).astype(jnp.float32)
    oh1 = (fl[:, 2:3] >= fl[:, 3:4]).astype(jnp.float32)

    m0 = jnp.maximum(fl[:, 0:1], fl[:, 1:2])
    e00 = jnp.exp(fl[:, 0:1] - m0)
    e01 = jnp.exp(fl[:, 1:2] - m0)
    p00 = e00 / (e00 + e01)
    m1 = jnp.maximum(fl[:, 2:3], fl[:, 3:4])
    e10 = jnp.exp(fl[:, 2:3] - m1)
    e11 = jnp.exp(fl[:, 3:4] - m1)
    p10 = e10 / (e10 + e11)

    c0 = cw[:, 0:1] * oh0
    c1 = cw[:, 0:1] * (1.0 - oh0)
    c2 = cw[:, 1:2] * oh1
    c3 = cw[:, 1:2] * (1.0 - oh1)
    den = (cw[:, 0:1] + cw[:, 1:2]) + 1e-8
    leaf_ref[...] = jnp.concatenate([c0, c1, c2, c3], axis=1) / den

    @pl.when(i == 0)
    def _init():
        for j in range(8):
            acc_ref[j] = 0.0

    acc_ref[0] += jnp.sum(ohc0)
    acc_ref[1] += jnp.sum(cw[:, 0:1])
    acc_ref[2] += jnp.sum(oh0)
    acc_ref[3] += jnp.sum(p00)
    acc_ref[4] += jnp.sum(oh1)
    acc_ref[5] += jnp.sum(p10)

    @pl.when(i == 0)
    def _zero_aux():
        aux_ref[...] = jnp.zeros((1, 1), jnp.float32)

    @pl.when(i == NT - 1)
    def _finalize():
        nb = jnp.float32(B)
        fc0 = acc_ref[0] / nb
        pc0 = acc_ref[1] / nb
        aux_c = 2.0 * (fc0 * pc0 + (1.0 - fc0) * (1.0 - pc0))
        f00 = acc_ref[2] / nb
        p00s = acc_ref[3] / nb
        f10 = acc_ref[4] / nb
        p10s = acc_ref[5] / nb
        aux_f = (2.0 * (f00 * p00s + (1.0 - f00) * (1.0 - p00s))
                 + 2.0 * (f10 * p10s + (1.0 - f10) * (1.0 - p10s)))
        aux_ref[...] = (AUX_COEF * (aux_c + aux_f / N_SUPER)).reshape(1, 1)


def _expert_body(leaf_ref, x_ref, w1_ref, b1_ref, w2_ref, b2_ref,
                 g_ref, beta_ref, rhw_ref, chw_ref, rhb_ref, chb_ref,
                 price_ref, dir_ref, pacc_ref, dacc_ref):
    e = pl.program_id(0)
    i = pl.program_id(1)
    x = x_ref[...]
    hh = lax.dot_general(x, w1_ref[0], (((1,), (1,)), ((), ())),
                         preferred_element_type=jnp.float32)
    hh = hh + b1_ref[0]
    hh = 0.5 * hh * (1.0 + lax.erf(hh * (1.0 / math.sqrt(2.0))))
    eo = lax.dot_general(hh, w2_ref[0], (((1,), (1,)), ((), ())),
                         preferred_element_type=jnp.float32)
    eo = eo + b2_ref[0]
    mu = jnp.mean(eo, axis=1, keepdims=True)
    d = eo - mu
    var = jnp.mean(d * d, axis=1, keepdims=True)
    rstd = lax.rsqrt(var + 1e-5)
    eon = d * rstd * g_ref[0] + beta_ref[0]
    pr = lax.dot_general(eon, rhw_ref[...], (((1,), (1,)), ((), ())),
                         preferred_element_type=jnp.float32)
    dr = lax.dot_general(eon, chw_ref[...], (((1,), (1,)), ((), ())),
                         preferred_element_type=jnp.float32)
    lane = lax.broadcasted_iota(jnp.int32, (1, E), 1)
    l = jnp.sum(jnp.where(lane == e, leaf_ref[...], 0.0), axis=1,
                keepdims=True)
    cp = l * pr
    cd = l * dr
    sl = pl.ds(i * BT, BT)

    @pl.when(e == 0)
    def _init():
        pacc_ref[sl, :] = cp
        dacc_ref[sl, :] = cd

    @pl.when(e > 0)
    def _acc():
        pacc_ref[sl, :] += cp
        dacc_ref[sl, :] += cd

    price_ref[...] = pacc_ref[sl, :] + rhb_ref[...]
    dir_ref[...] = 1.0 / (1.0 + jnp.exp(-(dacc_ref[sl, :] + chb_ref[...])))


@jax.jit
def kernel(x, cg_w1, cg_b1, cg_w2, cg_b2, fg_w, fg_b, ex_w1, ex_b1,
           ex_w2, ex_b2, ex_g, ex_beta, rh_w, rh_b, ch_w, ch_b):
    f32 = jnp.float32
    fg_w2d = fg_w.reshape(E, IN_DIM + N_SUPER)
    leaf, aux = pl.pallas_call(
        _gating_body,
        grid=(NT,),
        in_specs=[
            pl.BlockSpec((BT, IN_DIM), lambda i: (i, 0)),
            pl.BlockSpec((IN_DIM // 2, IN_DIM), lambda i: (0, 0)),
            pl.BlockSpec((1, IN_DIM // 2), lambda i: (0, 0)),
            pl.BlockSpec((N_SUPER, IN_DIM // 2), lambda i: (0, 0)),
            pl.BlockSpec((1, N_SUPER), lambda i: (0, 0)),
            pl.BlockSpec((E, IN_DIM + N_SUPER), lambda i: (0, 0)),
            pl.BlockSpec((1, E), lambda i: (0, 0)),
        ],
        out_specs=[
            pl.BlockSpec((BT, E), lambda i: (i, 0)),
            pl.BlockSpec((1, 1), lambda i: (0, 0)),
        ],
        out_shape=[
            jax.ShapeDtypeStruct((B, E), f32),
            jax.ShapeDtypeStruct((1, 1), f32),
        ],
        scratch_shapes=[pltpu.SMEM((8,), f32)],
    )(x, cg_w1, cg_b1.reshape(1, -1), cg_w2, cg_b2.reshape(1, -1),
      fg_w2d, fg_b.reshape(1, E))

    price, direction = pl.pallas_call(
        _expert_body,
        grid=(E, NT),
        in_specs=[
            pl.BlockSpec((BT, E), lambda e, i: (i, 0)),
            pl.BlockSpec((BT, IN_DIM), lambda e, i: (i, 0)),
            pl.BlockSpec((1, HID, IN_DIM), lambda e, i: (e, 0, 0)),
            pl.BlockSpec((1, 1, HID), lambda e, i: (e, 0, 0)),
            pl.BlockSpec((1, OUT, HID), lambda e, i: (e, 0, 0)),
            pl.BlockSpec((1, 1, OUT), lambda e, i: (e, 0, 0)),
            pl.BlockSpec((1, 1, OUT), lambda e, i: (e, 0, 0)),
            pl.BlockSpec((1, 1, OUT), lambda e, i: (e, 0, 0)),
            pl.BlockSpec((1, OUT), lambda e, i: (0, 0)),
            pl.BlockSpec((1, OUT), lambda e, i: (0, 0)),
            pl.BlockSpec((1, 1), lambda e, i: (0, 0)),
            pl.BlockSpec((1, 1), lambda e, i: (0, 0)),
        ],
        out_specs=[
            pl.BlockSpec((BT, 1), lambda e, i: (i, 0)),
            pl.BlockSpec((BT, 1), lambda e, i: (i, 0)),
        ],
        out_shape=[
            jax.ShapeDtypeStruct((B, 1), f32),
            jax.ShapeDtypeStruct((B, 1), f32),
        ],
        scratch_shapes=[
            pltpu.VMEM((B, 1), f32),
            pltpu.VMEM((B, 1), f32),
        ],
    )(leaf, x, ex_w1, ex_b1.reshape(E, 1, HID), ex_w2,
      ex_b2.reshape(E, 1, OUT), ex_g.reshape(E, 1, OUT),
      ex_beta.reshape(E, 1, OUT),
      rh_w, ch_w, rh_b.reshape(1, 1), ch_b.reshape(1, 1))

    return price, direction, leaf, aux.reshape(())
